# baseline (speedup 1.0000x reference)
import functools

import jax
import jax.numpy as jnp
from jax import lax
from jax.experimental import pallas as pl
from jax.experimental.pallas import tpu as pltpu
from jax.experimental.pallas import tpu_sc as plsc


def _build_relayout(SI, dp, NC, NS):
    NW = NC * NS
    n_tiles = (SI + 127) // 128
    iters = (n_tiles + NW - 1) // NW
    mesh = plsc.VectorSubcoreMesh(core_axis_name="c", subcore_axis_name="s")

    @functools.partial(
        pl.kernel,
        mesh=mesh,
        compiler_params=pltpu.CompilerParams(use_tc_tiling_on_sc=True,
                                             needs_layout_passes=False),
        out_type=jax.ShapeDtypeStruct((n_tiles * 128 * dp,), jnp.float32),
        scratch_types=[
            pltpu.VMEM((2, 8, 128), jnp.float32),
            pltpu.VMEM((2, 8, 128), jnp.float32),
            pltpu.VMEM((128 * dp,), jnp.float32),
            pltpu.VMEM((128 * dp,), jnp.float32),
            pltpu.SemaphoreType.DMA,
            pltpu.SemaphoreType.DMA,
            pltpu.SemaphoreType.DMA,
            pltpu.SemaphoreType.DMA,
        ],
    )
    def ka(src_hbm, dst_hbm, in0, in1, out0, out1,
           sem_i0, sem_i1, sem_o0, sem_o1):
        wid = lax.axis_index("s") * NC + lax.axis_index("c")
        ins = (in0, in1)
        outs = (out0, out1)
        isems = (sem_i0, sem_i1)
        osems = (sem_o0, sem_o1)
        base16 = lax.shift_left(lax.iota(jnp.int32, 16), 4)

        def fire_in(t, b):
            c0 = t * 128
            pltpu.async_copy(src_hbm.at[pl.ds(0, 8), pl.ds(c0, 128)],
                             ins[b].at[0], isems[b])
            pltpu.async_copy(src_hbm.at[pl.ds(8, 8), pl.ds(c0, 128)],
                             ins[b].at[1], isems[b])

        def drain_in(t, b):
            c0 = t * 128
            pltpu.make_async_copy(src_hbm.at[pl.ds(0, 8), pl.ds(c0, 128)],
                                  ins[b].at[0], isems[b]).wait()
            pltpu.make_async_copy(src_hbm.at[pl.ds(8, 8), pl.ds(c0, 128)],
                                  ins[b].at[1], isems[b]).wait()

        @pl.when(wid < n_tiles)
        def _():
            fire_in(wid, 0)

        def body(j, carry):
            for b in range(2):
                i = j * 2 + b
                t = wid + i * NW
                tn = t + NW

                @pl.when(tn < n_tiles)
                def _():
                    fire_in(tn, 1 - b)

                @pl.when(t < n_tiles)
                def _():
                    drain_in(t, b)
                    buf_in, buf_out, sem_o = ins[b], outs[b], osems[b]

                    @pl.when(i >= 2)
                    def _():
                        pltpu.make_async_copy(
                            buf_out, dst_hbm.at[pl.ds(0, 128 * dp)],
                            sem_o).wait()

                    for g in range(2):
                        for r in range(8):
                            d_off = g * 8 + r
                            for k in range(8):
                                vec = buf_in[g, r, pl.ds(16 * k, 16)]
                                plsc.store_scatter(
                                    buf_out,
                                    [base16 + (256 * k + d_off)], vec)
                    pltpu.async_copy(
                        buf_out, dst_hbm.at[pl.ds(t * 128 * dp, 128 * dp)],
                        sem_o)
            return carry

        lax.fori_loop(0, (iters + 1) // 2, body, 0)
        for b in range(2):
            pltpu.make_async_copy(
                outs[b], dst_hbm.at[pl.ds(0, 128 * dp)], osems[b]).wait()

    return ka


def _build_gather(N, B, L, dp, ds, NC, NS):
    NW = NC * NS
    LG = L // 8
    NBT = B // 128
    s_per_w = B // NW
    GD = dp // 8

    mesh = plsc.VectorSubcoreMesh(core_axis_name="c", subcore_axis_name="s")

    @functools.partial(
        pl.kernel,
        mesh=mesh,
        compiler_params=pltpu.CompilerParams(use_tc_tiling_on_sc=False,
                                             needs_layout_passes=False),
        out_type=(
            jax.ShapeDtypeStruct((L * GD * NBT * 1024,), jnp.float32),
            jax.ShapeDtypeStruct((B, ds), jnp.float32),
        ),
        scratch_types=[
            pltpu.VMEM((8, 128), jnp.int32),
            pltpu.VMEM((8, 128), jnp.int32),
            pltpu.VMEM((1024, dp), jnp.float32),
            pltpu.VMEM((8 * GD * 1024,), jnp.float32),
            pltpu.VMEM((8 * GD * 1024,), jnp.float32),
            pltpu.VMEM((s_per_w,), jnp.int32),
            pltpu.VMEM((s_per_w, ds), jnp.float32),
            pltpu.SemaphoreType.DMA,
            pltpu.SemaphoreType.DMA,
            pltpu.SemaphoreType.DMA,
            pltpu.SemaphoreType.DMA,
        ],
    )
    def kb(idx_hbm, spk_hbm, lutp_hbm, luts_hbm, outp_hbm, outs_hbm,
           idx0_v, idx1_v, rows_v, xb0, xb1, sidx_v, srows_v,
           sem_i, sem_g, sem_s0, sem_s1):
        wid = lax.axis_index("s") * NC + lax.axis_index("c")
        idx_bufs = (idx0_v, idx1_v)
        xbufs = (xb0, xb1)
        store_sems = (sem_s0, sem_s1)
        iota16 = lax.iota(jnp.int32, 16)
        base128 = lax.shift_left(iota16, 7)
        unit_w = 8 * GD * 1024

        sbase = wid * s_per_w
        pltpu.sync_copy(spk_hbm.at[pl.ds(sbase, s_per_w)], sidx_v)
        pltpu.async_copy(luts_hbm.at[sidx_v], srows_v, sem_g).wait()
        pltpu.sync_copy(srows_v, outs_hbm.at[pl.ds(sbase, s_per_w)])

        pltpu.async_copy(idx_hbm.at[0, wid], idx0_v, sem_i)

        def unit(i, b):
            idx_v, xb, sem_s = idx_bufs[b], xbufs[b], store_sems[b]
            pltpu.make_async_copy(idx_hbm.at[0, wid], idx_v, sem_i).wait()
            hs = []
            for r in range(8):
                hs.append(pltpu.async_copy(
                    lutp_hbm.at[idx_v.at[r]],
                    rows_v.at[pl.ds(r * 128, 128)], sem_g))

            @pl.when(i + 1 < LG)
            def _():
                pltpu.async_copy(idx_hbm.at[i + 1, wid], idx_bufs[1 - b],
                                 sem_i)
            for h in hs:
                h.wait()

            @pl.when(i >= 2)
            def _():
                pltpu.make_async_copy(
                    xb, outp_hbm.at[pl.ds(0, unit_w)], sem_s).wait()

            def il_body(il, carry):
                row0 = il * 128
                for c in range(128):
                    vec = rows_v[row0 + c]
                    plsc.store_scatter(
                        xb, [base128 + (il * (GD * 1024) + c)], vec)
                return carry

            lax.fori_loop(0, 8, il_body, 0)
            for il in range(8):
                for g in range(GD):
                    off = (((i * 8 + il) * GD + g) * NBT + wid) * 1024
                    pltpu.async_copy(
                        xb.at[pl.ds((il * GD + g) * 1024, 1024)],
                        outp_hbm.at[pl.ds(off, 1024)], sem_s)

        def body(j, carry):
            for b in range(2):
                unit(j * 2 + b, b)
            return carry

        lax.fori_loop(0, LG // 2, body, 0)
        if LG % 2:
            unit(jnp.int32(LG - 1), 0)
        for b in range(2):
            pltpu.make_async_copy(
                xbufs[b], outp_hbm.at[pl.ds(0, unit_w)],
                store_sems[b]).wait()

    return kb


def kernel(sentence, speaker_id, Lut_P, Lut_S):
    B, L = sentence.shape
    SI, dp = Lut_P.shape
    ds = Lut_S.shape[1]
    N = B * L

    info = plsc.get_sparse_core_info()
    NC, NS = info.num_cores, info.num_subcores

    ka = _build_relayout(SI, dp, NC, NS)
    si_pad = ((SI + 127) // 128) * 128
    table_rm = ka(Lut_P.T).reshape(si_pad, dp)

    kb = _build_gather(N, B, L, dp, ds, NC, NS)
    idx4 = (sentence.T.astype(jnp.int32)
            .reshape(L // 8, 8, B // 128, 128)
            .transpose(0, 2, 1, 3))
    spk_flat = speaker_id.reshape(B).astype(jnp.int32)
    outp5, outs = kb(idx4, spk_flat, table_rm, Lut_S)
    outp = (outp5.reshape(L, dp // 8, B // 128, 8, 128)
            .transpose(2, 4, 0, 1, 3).reshape(B, L, dp))
    return outp, outs


# speedup vs baseline: 1.1034x; 1.1034x over previous
import functools

import jax
import jax.numpy as jnp
from jax import lax
from jax.experimental import pallas as pl
from jax.experimental.pallas import tpu as pltpu
from jax.experimental.pallas import tpu_sc as plsc


def _build_relayout(SI, dp, NC, NS):
    NW = NC * NS
    n_tiles = (SI + 127) // 128
    iters = (n_tiles + NW - 1) // NW
    mesh = plsc.VectorSubcoreMesh(core_axis_name="c", subcore_axis_name="s")

    @functools.partial(
        pl.kernel,
        mesh=mesh,
        compiler_params=pltpu.CompilerParams(use_tc_tiling_on_sc=True,
                                             needs_layout_passes=False),
        out_type=jax.ShapeDtypeStruct((n_tiles * 128 * dp,), jnp.float32),
        scratch_types=[
            pltpu.VMEM((2, 8, 128), jnp.float32),
            pltpu.VMEM((2, 8, 128), jnp.float32),
            pltpu.VMEM((128 * dp,), jnp.float32),
            pltpu.VMEM((128 * dp,), jnp.float32),
            pltpu.SemaphoreType.DMA,
            pltpu.SemaphoreType.DMA,
            pltpu.SemaphoreType.DMA,
            pltpu.SemaphoreType.DMA,
        ],
    )
    def ka(src_hbm, dst_hbm, in0, in1, out0, out1,
           sem_i0, sem_i1, sem_o0, sem_o1):
        wid = lax.axis_index("s") * NC + lax.axis_index("c")
        ins = (in0, in1)
        outs = (out0, out1)
        isems = (sem_i0, sem_i1)
        osems = (sem_o0, sem_o1)
        base16 = lax.shift_left(lax.iota(jnp.int32, 16), 4)

        def fire_in(t, b):
            c0 = t * 128
            pltpu.async_copy(src_hbm.at[pl.ds(0, 8), pl.ds(c0, 128)],
                             ins[b].at[0], isems[b])
            pltpu.async_copy(src_hbm.at[pl.ds(8, 8), pl.ds(c0, 128)],
                             ins[b].at[1], isems[b])

        def drain_in(t, b):
            c0 = t * 128
            pltpu.make_async_copy(src_hbm.at[pl.ds(0, 8), pl.ds(c0, 128)],
                                  ins[b].at[0], isems[b]).wait()
            pltpu.make_async_copy(src_hbm.at[pl.ds(8, 8), pl.ds(c0, 128)],
                                  ins[b].at[1], isems[b]).wait()

        @pl.when(wid < n_tiles)
        def _():
            fire_in(wid, 0)

        def body(j, carry):
            for b in range(2):
                i = j * 2 + b
                t = wid + i * NW
                tn = t + NW

                @pl.when(tn < n_tiles)
                def _():
                    fire_in(tn, 1 - b)

                @pl.when(t < n_tiles)
                def _():
                    drain_in(t, b)
                    buf_in, buf_out, sem_o = ins[b], outs[b], osems[b]

                    @pl.when(i >= 2)
                    def _():
                        pltpu.make_async_copy(
                            buf_out, dst_hbm.at[pl.ds(0, 128 * dp)],
                            sem_o).wait()

                    for g in range(2):
                        for r in range(8):
                            d_off = g * 8 + r
                            for k in range(8):
                                vec = buf_in[g, r, pl.ds(16 * k, 16)]
                                plsc.store_scatter(
                                    buf_out,
                                    [base16 + (256 * k + d_off)], vec)
                    pltpu.async_copy(
                        buf_out, dst_hbm.at[pl.ds(t * 128 * dp, 128 * dp)],
                        sem_o)
            return carry

        lax.fori_loop(0, (iters + 1) // 2, body, 0)
        for b in range(2):
            pltpu.make_async_copy(
                outs[b], dst_hbm.at[pl.ds(0, 128 * dp)], osems[b]).wait()

    return ka


def _build_gather(N, B, L, dp, ds, NC, NS):
    NW = NC * NS
    LG = L // 8
    NBT = B // 128
    s_per_w = B // NW
    GD = dp // 8

    mesh = plsc.VectorSubcoreMesh(core_axis_name="c", subcore_axis_name="s")

    @functools.partial(
        pl.kernel,
        mesh=mesh,
        compiler_params=pltpu.CompilerParams(use_tc_tiling_on_sc=False,
                                             needs_layout_passes=False),
        out_type=(
            jax.ShapeDtypeStruct((L, GD, NBT, 8, 128), jnp.float32),
            jax.ShapeDtypeStruct((B, ds), jnp.float32),
        ),
        scratch_types=[
            pltpu.VMEM((8, 128), jnp.int32),
            pltpu.VMEM((8, 128), jnp.int32),
            pltpu.VMEM((1024, dp), jnp.float32),
            pltpu.VMEM((8, GD, 8, 128), jnp.float32),
            pltpu.VMEM((8, GD, 8, 128), jnp.float32),
            pltpu.VMEM((s_per_w,), jnp.int32),
            pltpu.VMEM((s_per_w, ds), jnp.float32),
            pltpu.SemaphoreType.DMA,
            pltpu.SemaphoreType.DMA,
            pltpu.SemaphoreType.DMA,
            pltpu.SemaphoreType.DMA,
        ],
    )
    def kb(idx_hbm, spk_hbm, lutp_hbm, luts_hbm, outp_hbm, outs_hbm,
           idx0_v, idx1_v, rows_v, xb0, xb1, sidx_v, srows_v,
           sem_i, sem_g, sem_s0, sem_s1):
        wid = lax.axis_index("s") * NC + lax.axis_index("c")
        idx_bufs = (idx0_v, idx1_v)
        xbufs = (xb0, xb1)
        store_sems = (sem_s0, sem_s1)
        iota16 = lax.iota(jnp.int32, 16)
        base128 = lax.shift_left(iota16, 7)
        unit_w = 8 * GD * 1024

        sbase = wid * s_per_w
        pltpu.sync_copy(spk_hbm.at[pl.ds(sbase, s_per_w)], sidx_v)
        pltpu.async_copy(luts_hbm.at[sidx_v], srows_v, sem_g).wait()
        pltpu.sync_copy(srows_v, outs_hbm.at[pl.ds(sbase, s_per_w)])

        pltpu.async_copy(idx_hbm.at[0, wid], idx0_v, sem_i)

        def unit(i, b):
            idx_v, xb, sem_s = idx_bufs[b], xbufs[b], store_sems[b]
            pltpu.make_async_copy(idx_hbm.at[0, wid], idx_v, sem_i).wait()
            hs = []
            for r in range(8):
                hs.append(pltpu.async_copy(
                    lutp_hbm.at[idx_v.at[r]],
                    rows_v.at[pl.ds(r * 128, 128)], sem_g))

            @pl.when(i + 1 < LG)
            def _():
                pltpu.async_copy(idx_hbm.at[i + 1, wid], idx_bufs[1 - b],
                                 sem_i)
            for h in hs:
                h.wait()

            @pl.when(i >= 2)
            def _():
                pltpu.make_async_copy(
                    xb, outp_hbm.at[pl.ds(0, 8), :, 0], sem_s).wait()

            def il_body(il, carry):
                row0 = il * 128
                for g in range(GD):
                    for r in range(8):
                        d_vec = jnp.full((16,), g * 8 + r, jnp.int32)
                        for k in range(8):
                            vec = plsc.load_gather(
                                rows_v,
                                [iota16 + (row0 + 16 * k), d_vec])
                            xb[il, g, r, pl.ds(16 * k, 16)] = vec
                return carry

            lax.fori_loop(0, 8, il_body, 0)
            pltpu.async_copy(xb, outp_hbm.at[pl.ds(i * 8, 8), :, wid], sem_s)

        def body(j, carry):
            for b in range(2):
                unit(j * 2 + b, b)
            return carry

        lax.fori_loop(0, LG // 2, body, 0)
        if LG % 2:
            unit(jnp.int32(LG - 1), 0)
        for b in range(2):
            pltpu.make_async_copy(
                xbufs[b], outp_hbm.at[pl.ds(0, 8), :, 0],
                store_sems[b]).wait()

    return kb


def kernel(sentence, speaker_id, Lut_P, Lut_S):
    B, L = sentence.shape
    SI, dp = Lut_P.shape
    ds = Lut_S.shape[1]
    N = B * L

    info = plsc.get_sparse_core_info()
    NC, NS = info.num_cores, info.num_subcores

    ka = _build_relayout(SI, dp, NC, NS)
    si_pad = ((SI + 127) // 128) * 128
    table_rm = ka(Lut_P.T).reshape(si_pad, dp)

    kb = _build_gather(N, B, L, dp, ds, NC, NS)
    idx4 = (sentence.T.astype(jnp.int32)
            .reshape(L // 8, 8, B // 128, 128)
            .transpose(0, 2, 1, 3))
    spk_flat = speaker_id.reshape(B).astype(jnp.int32)
    outp5, outs = kb(idx4, spk_flat, table_rm, Lut_S)
    outp = outp5.transpose(2, 4, 0, 1, 3).reshape(B, L, dp)
    return outp, outs


# speedup vs baseline: 1.2496x; 1.1324x over previous
import functools

import jax
import jax.numpy as jnp
from jax import lax
from jax.experimental import pallas as pl
from jax.experimental.pallas import tpu as pltpu
from jax.experimental.pallas import tpu_sc as plsc


def _build_relayout(SI, dp, NC, NS):
    NW = NC * NS
    n_tiles = (SI + 127) // 128
    iters = (n_tiles + NW - 1) // NW
    mesh = plsc.VectorSubcoreMesh(core_axis_name="c", subcore_axis_name="s")

    @functools.partial(
        pl.kernel,
        mesh=mesh,
        compiler_params=pltpu.CompilerParams(use_tc_tiling_on_sc=True,
                                             needs_layout_passes=False),
        out_type=jax.ShapeDtypeStruct((n_tiles * 128 * dp,), jnp.float32),
        scratch_types=[
            pltpu.VMEM((2, 8, 128), jnp.float32),
            pltpu.VMEM((2, 8, 128), jnp.float32),
            pltpu.VMEM((128 * dp,), jnp.float32),
            pltpu.VMEM((128 * dp,), jnp.float32),
            pltpu.SemaphoreType.DMA,
            pltpu.SemaphoreType.DMA,
            pltpu.SemaphoreType.DMA,
            pltpu.SemaphoreType.DMA,
        ],
    )
    def ka(src_hbm, dst_hbm, in0, in1, out0, out1,
           sem_i0, sem_i1, sem_o0, sem_o1):
        wid = lax.axis_index("s") * NC + lax.axis_index("c")
        ins = (in0, in1)
        outs = (out0, out1)
        isems = (sem_i0, sem_i1)
        osems = (sem_o0, sem_o1)
        base16 = lax.shift_left(lax.iota(jnp.int32, 16), 4)

        def fire_in(t, b):
            c0 = t * 128
            pltpu.async_copy(src_hbm.at[pl.ds(0, 8), pl.ds(c0, 128)],
                             ins[b].at[0], isems[b])
            pltpu.async_copy(src_hbm.at[pl.ds(8, 8), pl.ds(c0, 128)],
                             ins[b].at[1], isems[b])

        def drain_in(t, b):
            c0 = t * 128
            pltpu.make_async_copy(src_hbm.at[pl.ds(0, 8), pl.ds(c0, 128)],
                                  ins[b].at[0], isems[b]).wait()
            pltpu.make_async_copy(src_hbm.at[pl.ds(8, 8), pl.ds(c0, 128)],
                                  ins[b].at[1], isems[b]).wait()

        @pl.when(wid < n_tiles)
        def _():
            fire_in(wid, 0)

        def body(j, carry):
            for b in range(2):
                i = j * 2 + b
                t = wid + i * NW
                tn = t + NW

                @pl.when(tn < n_tiles)
                def _():
                    fire_in(tn, 1 - b)

                @pl.when(t < n_tiles)
                def _():
                    drain_in(t, b)
                    buf_in, buf_out, sem_o = ins[b], outs[b], osems[b]

                    @pl.when(i >= 2)
                    def _():
                        pltpu.make_async_copy(
                            buf_out, dst_hbm.at[pl.ds(0, 128 * dp)],
                            sem_o).wait()

                    for g in range(2):
                        for r in range(8):
                            d_off = g * 8 + r
                            for k in range(8):
                                vec = buf_in[g, r, pl.ds(16 * k, 16)]
                                plsc.store_scatter(
                                    buf_out,
                                    [base16 + (256 * k + d_off)], vec)
                    pltpu.async_copy(
                        buf_out, dst_hbm.at[pl.ds(t * 128 * dp, 128 * dp)],
                        sem_o)
            return carry

        lax.fori_loop(0, (iters + 1) // 2, body, 0)
        for b in range(2):
            pltpu.make_async_copy(
                outs[b], dst_hbm.at[pl.ds(0, 128 * dp)], osems[b]).wait()

    return ka


def _build_gather(N, B, L, dp, ds, NC, NS):
    NW = NC * NS
    LG = L // 8
    NBT = B // 128
    s_per_w = B // NW
    GD = dp // 8

    mesh = plsc.VectorSubcoreMesh(core_axis_name="c", subcore_axis_name="s")

    @functools.partial(
        pl.kernel,
        mesh=mesh,
        compiler_params=pltpu.CompilerParams(use_tc_tiling_on_sc=False,
                                             needs_layout_passes=False),
        out_type=(
            jax.ShapeDtypeStruct((L, GD, NBT, 8, 128), jnp.float32),
            jax.ShapeDtypeStruct((B, ds), jnp.float32),
        ),
        scratch_types=[
            pltpu.VMEM((8, 128), jnp.int32),
            pltpu.VMEM((8, 128), jnp.int32),
            pltpu.VMEM((1024, dp), jnp.float32),
            pltpu.VMEM((1024, dp), jnp.float32),
            pltpu.VMEM((8, GD, 8, 128), jnp.float32),
            pltpu.VMEM((8, GD, 8, 128), jnp.float32),
            pltpu.VMEM((s_per_w,), jnp.int32),
            pltpu.VMEM((s_per_w, ds), jnp.float32),
            pltpu.SemaphoreType.DMA,
            pltpu.SemaphoreType.DMA,
            pltpu.SemaphoreType.DMA,
            pltpu.SemaphoreType.DMA,
        ],
    )
    def kb(idx_hbm, spk_hbm, lutp_hbm, luts_hbm, outp_hbm, outs_hbm,
           idx0_v, idx1_v, rows0_v, rows1_v, xb0, xb1, sidx_v, srows_v,
           sem_i, sem_g, sem_s0, sem_s1):
        wid = lax.axis_index("s") * NC + lax.axis_index("c")
        idx_bufs = (idx0_v, idx1_v)
        rows_bufs = (rows0_v, rows1_v)
        xbufs = (xb0, xb1)
        store_sems = (sem_s0, sem_s1)
        iota16 = lax.iota(jnp.int32, 16)

        sbase = wid * s_per_w
        pltpu.sync_copy(spk_hbm.at[pl.ds(sbase, s_per_w)], sidx_v)
        pltpu.async_copy(luts_hbm.at[sidx_v], srows_v, sem_g).wait()
        pltpu.sync_copy(srows_v, outs_hbm.at[pl.ds(sbase, s_per_w)])

        pltpu.async_copy(idx_hbm.at[0, wid], idx0_v, sem_i)

        def shuffle(i, b):
            rows_v, xb, sem_s = rows_bufs[b], xbufs[b], store_sems[b]

            @pl.when(i >= 2)
            def _():
                pltpu.make_async_copy(
                    xb, outp_hbm.at[pl.ds(0, 8), :, 0], sem_s).wait()

            def il_body(il, carry):
                row0 = il * 128
                for g in range(GD):
                    for r in range(8):
                        d_vec = jnp.full((16,), g * 8 + r, jnp.int32)
                        for k in range(8):
                            vec = plsc.load_gather(
                                rows_v,
                                [iota16 + (row0 + 16 * k), d_vec])
                            xb[il, g, r, pl.ds(16 * k, 16)] = vec
                return carry

            lax.fori_loop(0, 8, il_body, 0)
            pltpu.async_copy(xb, outp_hbm.at[pl.ds(i * 8, 8), :, wid], sem_s)

        def stage(i, b):
            @pl.when(i < LG)
            def _():
                idx_v = idx_bufs[b]
                pltpu.make_async_copy(idx_hbm.at[0, wid], idx_v, sem_i).wait()
                for r in range(8):
                    pltpu.async_copy(
                        lutp_hbm.at[idx_v.at[r]],
                        rows_bufs[b].at[pl.ds(r * 128, 128)], sem_g)

                @pl.when(i + 1 < LG)
                def _():
                    pltpu.async_copy(idx_hbm.at[i + 1, wid], idx_bufs[1 - b],
                                     sem_i)

            @pl.when(i >= 1)
            def _():
                shuffle(i - 1, 1 - b)

            @pl.when(i < LG)
            def _():
                for r in range(8):
                    pltpu.make_async_copy(
                        lutp_hbm.at[idx_bufs[b].at[r]],
                        rows_bufs[b].at[pl.ds(r * 128, 128)], sem_g).wait()

        def body(j, carry):
            for b in range(2):
                stage(j * 2 + b, b)
            return carry

        lax.fori_loop(0, (LG + 2) // 2, body, 0)
        for b in range(2):
            pltpu.make_async_copy(
                xbufs[b], outp_hbm.at[pl.ds(0, 8), :, 0],
                store_sems[b]).wait()

    return kb


def kernel(sentence, speaker_id, Lut_P, Lut_S):
    B, L = sentence.shape
    SI, dp = Lut_P.shape
    ds = Lut_S.shape[1]
    N = B * L

    info = plsc.get_sparse_core_info()
    NC, NS = info.num_cores, info.num_subcores

    ka = _build_relayout(SI, dp, NC, NS)
    si_pad = ((SI + 127) // 128) * 128
    table_rm = ka(Lut_P.T).reshape(si_pad, dp)

    kb = _build_gather(N, B, L, dp, ds, NC, NS)
    idx4 = (sentence.T.astype(jnp.int32)
            .reshape(L // 8, 8, B // 128, 128)
            .transpose(0, 2, 1, 3))
    spk_flat = speaker_id.reshape(B).astype(jnp.int32)
    outp5, outs = kb(idx4, spk_flat, table_rm, Lut_S)
    outp = outp5.transpose(2, 4, 0, 1, 3).reshape(B, L, dp)
    return outp, outs


# speedup vs baseline: 1.3254x; 1.0607x over previous
import functools

import jax
import jax.numpy as jnp
from jax import lax
from jax.experimental import pallas as pl
from jax.experimental.pallas import tpu as pltpu
from jax.experimental.pallas import tpu_sc as plsc


def _build_relayout(SI, dp, NC, NS):
    NW = NC * NS
    n_tiles = (SI + 127) // 128
    iters = (n_tiles + NW - 1) // NW
    mesh = plsc.VectorSubcoreMesh(core_axis_name="c", subcore_axis_name="s")

    @functools.partial(
        pl.kernel,
        mesh=mesh,
        compiler_params=pltpu.CompilerParams(use_tc_tiling_on_sc=True,
                                             needs_layout_passes=False),
        out_type=jax.ShapeDtypeStruct((n_tiles * 128 * dp,), jnp.float32),
        scratch_types=[
            pltpu.VMEM((dp, 128), jnp.float32),
            pltpu.VMEM((dp, 128), jnp.float32),
            pltpu.VMEM((dp, 128), jnp.float32),
            pltpu.VMEM((128 * dp,), jnp.float32),
            pltpu.VMEM((128 * dp,), jnp.float32),
            pltpu.VMEM((128 * dp,), jnp.float32),
            pltpu.SemaphoreType.DMA,
            pltpu.SemaphoreType.DMA,
            pltpu.SemaphoreType.DMA,
            pltpu.SemaphoreType.DMA,
            pltpu.SemaphoreType.DMA,
            pltpu.SemaphoreType.DMA,
        ],
    )
    def ka(src_hbm, dst_hbm, in0, in1, in2, out0, out1, out2,
           sem_i0, sem_i1, sem_i2, sem_o0, sem_o1, sem_o2):
        wid = lax.axis_index("s") * NC + lax.axis_index("c")
        ins = (in0, in1, in2)
        outs = (out0, out1, out2)
        isems = (sem_i0, sem_i1, sem_i2)
        osems = (sem_o0, sem_o1, sem_o2)
        base16 = lax.shift_left(lax.iota(jnp.int32, 16), 4)

        def fire_in(t, b):
            pltpu.async_copy(
                src_hbm.at[pl.ds(0, dp), pl.ds(t * 128, 128)],
                ins[b], isems[b])

        def drain_in(t, b):
            pltpu.make_async_copy(
                src_hbm.at[pl.ds(0, dp), pl.ds(t * 128, 128)],
                ins[b], isems[b]).wait()

        for p in range(2):
            @pl.when(wid + p * NW < n_tiles)
            def _():
                fire_in(wid + p * NW, p)

        def body(j, carry):
            for b in range(3):
                i = j * 3 + b
                t = wid + i * NW
                tn = t + 2 * NW

                @pl.when(tn < n_tiles)
                def _():
                    fire_in(tn, (b + 2) % 3)

                @pl.when(t < n_tiles)
                def _():
                    drain_in(t, b)
                    buf_in, buf_out, sem_o = ins[b], outs[b], osems[b]

                    @pl.when(i >= 3)
                    def _():
                        pltpu.make_async_copy(
                            buf_out, dst_hbm.at[pl.ds(0, 128 * dp)],
                            sem_o).wait()

                    for d_off in range(dp):
                        for k in range(8):
                            vec = buf_in[d_off, pl.ds(16 * k, 16)]
                            plsc.store_scatter(
                                buf_out,
                                [base16 + (16 * dp * k + d_off)], vec)
                    pltpu.async_copy(
                        buf_out, dst_hbm.at[pl.ds(t * 128 * dp, 128 * dp)],
                        sem_o)
            return carry

        lax.fori_loop(0, (iters + 2) // 3, body, 0)
        for b in range(3):
            pltpu.make_async_copy(
                outs[b], dst_hbm.at[pl.ds(0, 128 * dp)], osems[b]).wait()

    return ka


def _build_gather(N, B, L, dp, ds, NC, NS):
    NW = NC * NS
    LG = L // 8
    NBT = B // 128
    s_per_w = B // NW
    GD = dp // 8

    mesh = plsc.VectorSubcoreMesh(core_axis_name="c", subcore_axis_name="s")

    @functools.partial(
        pl.kernel,
        mesh=mesh,
        compiler_params=pltpu.CompilerParams(use_tc_tiling_on_sc=False,
                                             needs_layout_passes=False),
        out_type=(
            jax.ShapeDtypeStruct((L, GD, NBT, 8, 128), jnp.float32),
            jax.ShapeDtypeStruct((B, ds), jnp.float32),
        ),
        scratch_types=[
            pltpu.VMEM((8, 128), jnp.int32),
            pltpu.VMEM((8, 128), jnp.int32),
            pltpu.VMEM((1024, dp), jnp.float32),
            pltpu.VMEM((1024, dp), jnp.float32),
            pltpu.VMEM((8, GD, 8, 128), jnp.float32),
            pltpu.VMEM((8, GD, 8, 128), jnp.float32),
            pltpu.VMEM((s_per_w,), jnp.int32),
            pltpu.VMEM((s_per_w, ds), jnp.float32),
            pltpu.SemaphoreType.DMA,
            pltpu.SemaphoreType.DMA,
            pltpu.SemaphoreType.DMA,
            pltpu.SemaphoreType.DMA,
        ],
    )
    def kb(idx_hbm, spk_hbm, lutp_hbm, luts_hbm, outp_hbm, outs_hbm,
           idx0_v, idx1_v, rows0_v, rows1_v, xb0, xb1, sidx_v, srows_v,
           sem_i, sem_g, sem_s0, sem_s1):
        wid = lax.axis_index("s") * NC + lax.axis_index("c")
        idx_bufs = (idx0_v, idx1_v)
        rows_bufs = (rows0_v, rows1_v)
        xbufs = (xb0, xb1)
        store_sems = (sem_s0, sem_s1)
        iota16 = lax.iota(jnp.int32, 16)

        sbase = wid * s_per_w
        pltpu.sync_copy(spk_hbm.at[pl.ds(sbase, s_per_w)], sidx_v)
        pltpu.async_copy(luts_hbm.at[sidx_v], srows_v, sem_g).wait()
        pltpu.sync_copy(srows_v, outs_hbm.at[pl.ds(sbase, s_per_w)])

        pltpu.async_copy(idx_hbm.at[0, wid], idx0_v, sem_i)

        def shuffle(i, b):
            rows_v, xb, sem_s = rows_bufs[b], xbufs[b], store_sems[b]

            @pl.when(i >= 2)
            def _():
                pltpu.make_async_copy(
                    xb, outp_hbm.at[pl.ds(0, 8), :, 0], sem_s).wait()

            def il_body(il, carry):
                row0 = il * 128
                for g in range(GD):
                    for r in range(8):
                        d_vec = jnp.full((16,), g * 8 + r, jnp.int32)
                        for k in range(8):
                            vec = plsc.load_gather(
                                rows_v,
                                [iota16 + (row0 + 16 * k), d_vec])
                            xb[il, g, r, pl.ds(16 * k, 16)] = vec
                return carry

            lax.fori_loop(0, 8, il_body, 0)
            pltpu.async_copy(xb, outp_hbm.at[pl.ds(i * 8, 8), :, wid], sem_s)

        def stage(i, b):
            @pl.when(i < LG)
            def _():
                idx_v = idx_bufs[b]
                pltpu.make_async_copy(idx_hbm.at[0, wid], idx_v, sem_i).wait()
                for r in range(8):
                    pltpu.async_copy(
                        lutp_hbm.at[idx_v.at[r]],
                        rows_bufs[b].at[pl.ds(r * 128, 128)], sem_g)

                @pl.when(i + 1 < LG)
                def _():
                    pltpu.async_copy(idx_hbm.at[i + 1, wid], idx_bufs[1 - b],
                                     sem_i)

            @pl.when(i >= 1)
            def _():
                shuffle(i - 1, 1 - b)

            @pl.when(i < LG)
            def _():
                for r in range(8):
                    pltpu.make_async_copy(
                        lutp_hbm.at[idx_bufs[b].at[r]],
                        rows_bufs[b].at[pl.ds(r * 128, 128)], sem_g).wait()

        def body(j, carry):
            for b in range(2):
                stage(j * 2 + b, b)
            return carry

        lax.fori_loop(0, (LG + 2) // 2, body, 0)
        for b in range(2):
            pltpu.make_async_copy(
                xbufs[b], outp_hbm.at[pl.ds(0, 8), :, 0],
                store_sems[b]).wait()

    return kb


def kernel(sentence, speaker_id, Lut_P, Lut_S):
    B, L = sentence.shape
    SI, dp = Lut_P.shape
    ds = Lut_S.shape[1]
    N = B * L

    info = plsc.get_sparse_core_info()
    NC, NS = info.num_cores, info.num_subcores

    ka = _build_relayout(SI, dp, NC, NS)
    si_pad = ((SI + 127) // 128) * 128
    table_rm = ka(Lut_P.T).reshape(si_pad, dp)

    kb = _build_gather(N, B, L, dp, ds, NC, NS)
    idx4 = (sentence.T.astype(jnp.int32)
            .reshape(L // 8, 8, B // 128, 128)
            .transpose(0, 2, 1, 3))
    spk_flat = speaker_id.reshape(B).astype(jnp.int32)
    outp5, outs = kb(idx4, spk_flat, table_rm, Lut_S)
    outp = outp5.transpose(2, 4, 0, 1, 3).reshape(B, L, dp)
    return outp, outs
